# baseline (speedup 1.0000x reference)
import functools

import jax
import jax.numpy as jnp
from jax import lax
from jax.experimental import pallas as pl
from jax.experimental.pallas import tpu as pltpu
from jax.experimental.pallas import tpu_sc as plsc

T = 128
H = 1024
I = 512
E = 64
K = 2

NC = 2
NS = 16
NW = NC * NS
LANES = 16
TOK_PER_W = T // NW
NEG_INF = float("-inf")


def _router_body(x_ref, rw_ref, out_ref):
    out_ref[...] = lax.dot_general(
        rw_ref[...], x_ref[...],
        dimension_numbers=(((1,), (1,)), ((), ())),
        precision=lax.Precision.HIGHEST,
        preferred_element_type=jnp.float32,
    )


def _router_logits(x, router_weight):
    return pl.pallas_call(
        _router_body,
        out_shape=jax.ShapeDtypeStruct((E, T), jnp.float32),
    )(x, router_weight)


NGRP = T // LANES


def _sc_routing_body(logits_hbm, scores_hbm, idx_hbm,
                     lg_v, s1_v, s2_v, i1_v, i2_v):
    wid = lax.axis_index("s") * NC + lax.axis_index("c")

    @pl.when(wid < NGRP)
    def _():
        pltpu.sync_copy(logits_hbm, lg_v)
        ninf = jnp.full((LANES,), NEG_INF, jnp.float32)
        m1 = ninf
        m2 = ninf
        i1 = jnp.zeros((LANES,), jnp.int32)
        i2 = jnp.zeros((LANES,), jnp.int32)
        col = wid * LANES
        for e in range(E):
            v = lg_v[e, pl.ds(col, LANES)]
            ev = jnp.full((LANES,), jnp.int32(e), jnp.int32)
            gt1 = v > m1
            gt2 = v > m2
            m2 = jnp.where(gt1, m1, jnp.where(gt2, v, m2))
            i2 = jnp.where(gt1, i1, jnp.where(gt2, ev, i2))
            m1 = jnp.where(gt1, v, m1)
            i1 = jnp.where(gt1, ev, i1)
        s1 = 1.0 / (1.0 + jnp.exp(m2 - m1))
        s1_v[...] = s1
        s2_v[...] = 1.0 - s1
        i1_v[...] = i1
        i2_v[...] = i2
        pltpu.sync_copy(s1_v, scores_hbm.at[0, pl.ds(col, LANES)])
        pltpu.sync_copy(s2_v, scores_hbm.at[1, pl.ds(col, LANES)])
        pltpu.sync_copy(i1_v, idx_hbm.at[0, pl.ds(col, LANES)])
        pltpu.sync_copy(i2_v, idx_hbm.at[1, pl.ds(col, LANES)])


def _sc_routing(logits_t):
    mesh = plsc.VectorSubcoreMesh(
        core_axis_name="c", subcore_axis_name="s",
        num_cores=NC, num_subcores=NS)
    f = pl.kernel(
        _sc_routing_body,
        out_type=(
            jax.ShapeDtypeStruct((K, T), jnp.float32),
            jax.ShapeDtypeStruct((K, T), jnp.int32),
        ),
        mesh=mesh,
        scratch_types=[
            pltpu.VMEM((E, T), jnp.float32),
            pltpu.VMEM((LANES,), jnp.float32),
            pltpu.VMEM((LANES,), jnp.float32),
            pltpu.VMEM((LANES,), jnp.int32),
            pltpu.VMEM((LANES,), jnp.int32),
        ],
    )
    return f(logits_t)


def _moe_body(x_ref, w1_ref, w2_ref, sc_ref, ix_ref, out_ref, xb_ref):
    e = pl.program_id(0)

    @pl.when(e == 0)
    def _():
        xb_ref[...] = x_ref[...].astype(jnp.bfloat16)

    xb = xb_ref[...]
    w1b = w1_ref[0].astype(jnp.bfloat16)
    gu = lax.dot_general(
        xb, w1b, dimension_numbers=(((1,), (1,)), ((), ())),
        preferred_element_type=jnp.float32)
    gate = gu[:, :I]
    up = gu[:, I:]
    h = (gate * jax.nn.sigmoid(gate)) * up
    hb = h.astype(jnp.bfloat16)
    w2b = w2_ref[0].astype(jnp.bfloat16)
    y = lax.dot_general(
        hb, w2b, dimension_numbers=(((1,), (1,)), ((), ())),
        preferred_element_type=jnp.float32)

    sel = jnp.where(ix_ref[...] == e, sc_ref[...], 0.0)
    ones = jnp.ones((K, 1), jnp.float32)
    wcol = lax.dot_general(
        sel, ones, dimension_numbers=(((0,), (0,)), ((), ())),
        precision=lax.Precision.HIGHEST,
        preferred_element_type=jnp.float32)
    contrib = y * wcol

    @pl.when(e == 0)
    def _():
        out_ref[...] = contrib

    @pl.when(e > 0)
    def _():
        out_ref[...] += contrib


def _moe(x, w1, w2, scores_t, idx_t):
    return pl.pallas_call(
        _moe_body,
        grid=(E,),
        in_specs=[
            pl.BlockSpec((T, H), lambda e: (0, 0)),
            pl.BlockSpec((1, 2 * I, H), lambda e: (e, 0, 0)),
            pl.BlockSpec((1, H, I), lambda e: (e, 0, 0)),
            pl.BlockSpec((K, T), lambda e: (0, 0)),
            pl.BlockSpec((K, T), lambda e: (0, 0)),
        ],
        out_specs=pl.BlockSpec((T, H), lambda e: (0, 0)),
        out_shape=jax.ShapeDtypeStruct((T, H), jnp.float32),
        scratch_shapes=[pltpu.VMEM((T, H), jnp.bfloat16)],
    )(x, w1, w2, scores_t, idx_t)


def kernel(hidden_states, router_weight, w1, w2):
    orig_shape = hidden_states.shape
    x = hidden_states.reshape(-1, orig_shape[-1])
    logits_t = _router_logits(x, router_weight)
    scores_t, idx_t = _sc_routing(logits_t)
    out = _moe(x, w1, w2, scores_t, idx_t)
    return out.reshape(orig_shape)


# speedup vs baseline: 1.0184x; 1.0184x over previous
import functools

import jax
import jax.numpy as jnp
from jax import lax
from jax.experimental import pallas as pl
from jax.experimental.pallas import tpu as pltpu
from jax.experimental.pallas import tpu_sc as plsc

T = 128
H = 1024
I = 512
E = 64
K = 2

NC = 2
NS = 16
NW = NC * NS
LANES = 16
TOK_PER_W = T // NW
NEG_INF = float("-inf")


def _router_body(x_ref, rw_ref, out_ref):
    out_ref[...] = lax.dot_general(
        rw_ref[...], x_ref[...],
        dimension_numbers=(((1,), (1,)), ((), ())),
        precision=lax.Precision.HIGHEST,
        preferred_element_type=jnp.float32,
    )


def _router_logits(x, router_weight):
    return pl.pallas_call(
        _router_body,
        out_shape=jax.ShapeDtypeStruct((E, T), jnp.float32),
    )(x, router_weight)


NGRP = T // LANES


def _sc_routing_body(logits_hbm, scores_hbm, idx_hbm,
                     lg_v, s1_v, s2_v, i1_v, i2_v):
    wid = lax.axis_index("s") * NC + lax.axis_index("c")

    @pl.when(wid < NGRP)
    def _():
        pltpu.sync_copy(logits_hbm, lg_v)
        ninf = jnp.full((LANES,), NEG_INF, jnp.float32)
        m1 = ninf
        m2 = ninf
        i1 = jnp.zeros((LANES,), jnp.int32)
        i2 = jnp.zeros((LANES,), jnp.int32)
        col = wid * LANES
        for e in range(E):
            v = lg_v[e, pl.ds(col, LANES)]
            ev = jnp.full((LANES,), jnp.int32(e), jnp.int32)
            gt1 = v > m1
            gt2 = v > m2
            m2 = jnp.where(gt1, m1, jnp.where(gt2, v, m2))
            i2 = jnp.where(gt1, i1, jnp.where(gt2, ev, i2))
            m1 = jnp.where(gt1, v, m1)
            i1 = jnp.where(gt1, ev, i1)
        s1 = 1.0 / (1.0 + jnp.exp(m2 - m1))
        s1_v[...] = s1
        s2_v[...] = 1.0 - s1
        i1_v[...] = i1
        i2_v[...] = i2
        pltpu.sync_copy(s1_v, scores_hbm.at[0, pl.ds(col, LANES)])
        pltpu.sync_copy(s2_v, scores_hbm.at[1, pl.ds(col, LANES)])
        pltpu.sync_copy(i1_v, idx_hbm.at[0, pl.ds(col, LANES)])
        pltpu.sync_copy(i2_v, idx_hbm.at[1, pl.ds(col, LANES)])


def _sc_routing(logits_t):
    mesh = plsc.VectorSubcoreMesh(
        core_axis_name="c", subcore_axis_name="s",
        num_cores=NC, num_subcores=NS)
    f = pl.kernel(
        _sc_routing_body,
        out_type=(
            jax.ShapeDtypeStruct((K, T), jnp.float32),
            jax.ShapeDtypeStruct((K, T), jnp.int32),
        ),
        mesh=mesh,
        scratch_types=[
            pltpu.VMEM((E, T), jnp.float32),
            pltpu.VMEM((LANES,), jnp.float32),
            pltpu.VMEM((LANES,), jnp.float32),
            pltpu.VMEM((LANES,), jnp.int32),
            pltpu.VMEM((LANES,), jnp.int32),
        ],
    )
    return f(logits_t)


def _moe_body(x_ref, w1_ref, w2_ref, sc_ref, ix_ref, out_ref, xb_ref):
    e = pl.program_id(0)

    @pl.when(e == 0)
    def _():
        xb_ref[...] = x_ref[...].astype(jnp.bfloat16)

    xb = xb_ref[...]
    w1b = w1_ref[0].astype(jnp.bfloat16)
    gate = lax.dot_general(
        xb, w1b[:I, :], dimension_numbers=(((1,), (1,)), ((), ())),
        preferred_element_type=jnp.float32)
    up = lax.dot_general(
        xb, w1b[I:, :], dimension_numbers=(((1,), (1,)), ((), ())),
        preferred_element_type=jnp.float32)
    h = (gate * jax.nn.sigmoid(gate)) * up
    hb = h.astype(jnp.bfloat16)
    w2b = w2_ref[0].astype(jnp.bfloat16)
    y = lax.dot_general(
        hb, w2b, dimension_numbers=(((1,), (1,)), ((), ())),
        preferred_element_type=jnp.float32)

    sel = jnp.where(ix_ref[...] == e, sc_ref[...], 0.0)
    selp = jnp.concatenate([sel, jnp.zeros((8 - K, T), jnp.float32)], axis=0)
    wcol = jnp.sum(jnp.transpose(selp), axis=1, keepdims=True)
    contrib = y * wcol

    @pl.when(e == 0)
    def _():
        out_ref[...] = contrib

    @pl.when(e > 0)
    def _():
        out_ref[...] += contrib


def _moe(x, w1, w2, scores_t, idx_t):
    return pl.pallas_call(
        _moe_body,
        grid=(E,),
        in_specs=[
            pl.BlockSpec((T, H), lambda e: (0, 0)),
            pl.BlockSpec((1, 2 * I, H), lambda e: (e, 0, 0)),
            pl.BlockSpec((1, H, I), lambda e: (e, 0, 0)),
            pl.BlockSpec((K, T), lambda e: (0, 0)),
            pl.BlockSpec((K, T), lambda e: (0, 0)),
        ],
        out_specs=pl.BlockSpec((T, H), lambda e: (0, 0)),
        out_shape=jax.ShapeDtypeStruct((T, H), jnp.float32),
        scratch_shapes=[pltpu.VMEM((T, H), jnp.bfloat16)],
    )(x, w1, w2, scores_t, idx_t)


def kernel(hidden_states, router_weight, w1, w2):
    orig_shape = hidden_states.shape
    x = hidden_states.reshape(-1, orig_shape[-1])
    logits_t = _router_logits(x, router_weight)
    scores_t, idx_t = _sc_routing(logits_t)
    out = _moe(x, w1, w2, scores_t, idx_t)
    return out.reshape(orig_shape)


# speedup vs baseline: 1.1359x; 1.1154x over previous
import functools

import jax
import jax.numpy as jnp
from jax import lax
from jax.experimental import pallas as pl
from jax.experimental.pallas import tpu as pltpu
from jax.experimental.pallas import tpu_sc as plsc

T = 128
H = 1024
I = 512
E = 64
K = 2
EPB = 2

NC = 2
NS = 16
NW = NC * NS
LANES = 16
TOK_PER_W = T // NW
NEG_INF = float("-inf")


def _router_body(x_ref, rw_ref, out_ref):
    out_ref[...] = lax.dot_general(
        rw_ref[...], x_ref[...],
        dimension_numbers=(((1,), (1,)), ((), ())),
        precision=lax.Precision.HIGHEST,
        preferred_element_type=jnp.float32,
    )


def _router_logits(x, router_weight):
    return pl.pallas_call(
        _router_body,
        out_shape=jax.ShapeDtypeStruct((E, T), jnp.float32),
    )(x, router_weight)


NGRP = T // LANES


def _sc_routing_body(logits_hbm, scores_hbm, idx_hbm,
                     lg_v, s1_v, s2_v, i1_v, i2_v):
    wid = lax.axis_index("s") * NC + lax.axis_index("c")

    @pl.when(wid < NGRP)
    def _():
        pltpu.sync_copy(logits_hbm, lg_v)
        ninf = jnp.full((LANES,), NEG_INF, jnp.float32)
        m1 = ninf
        m2 = ninf
        i1 = jnp.zeros((LANES,), jnp.int32)
        i2 = jnp.zeros((LANES,), jnp.int32)
        col = wid * LANES
        for e in range(E):
            v = lg_v[e, pl.ds(col, LANES)]
            ev = jnp.full((LANES,), jnp.int32(e), jnp.int32)
            gt1 = v > m1
            gt2 = v > m2
            m2 = jnp.where(gt1, m1, jnp.where(gt2, v, m2))
            i2 = jnp.where(gt1, i1, jnp.where(gt2, ev, i2))
            m1 = jnp.where(gt1, v, m1)
            i1 = jnp.where(gt1, ev, i1)
        s1 = 1.0 / (1.0 + jnp.exp(m2 - m1))
        s1_v[...] = s1
        s2_v[...] = 1.0 - s1
        i1_v[...] = i1
        i2_v[...] = i2
        pltpu.sync_copy(s1_v, scores_hbm.at[0, pl.ds(col, LANES)])
        pltpu.sync_copy(s2_v, scores_hbm.at[1, pl.ds(col, LANES)])
        pltpu.sync_copy(i1_v, idx_hbm.at[0, pl.ds(col, LANES)])
        pltpu.sync_copy(i2_v, idx_hbm.at[1, pl.ds(col, LANES)])


def _sc_routing(logits_t):
    mesh = plsc.VectorSubcoreMesh(
        core_axis_name="c", subcore_axis_name="s",
        num_cores=NC, num_subcores=NS)
    f = pl.kernel(
        _sc_routing_body,
        out_type=(
            jax.ShapeDtypeStruct((K, T), jnp.float32),
            jax.ShapeDtypeStruct((K, T), jnp.int32),
        ),
        mesh=mesh,
        scratch_types=[
            pltpu.VMEM((E, T), jnp.float32),
            pltpu.VMEM((LANES,), jnp.float32),
            pltpu.VMEM((LANES,), jnp.float32),
            pltpu.VMEM((LANES,), jnp.int32),
            pltpu.VMEM((LANES,), jnp.int32),
        ],
    )
    return f(logits_t)


def _moe_body(x_ref, w1_ref, w2_ref, sc_ref, ix_ref, out_ref, xb_ref):
    e = pl.program_id(0)

    @pl.when(e == 0)
    def _():
        xb_ref[...] = x_ref[...].astype(jnp.bfloat16)

    xb = xb_ref[...]
    contrib = jnp.zeros((T, H), jnp.float32)
    for j in range(EPB):
        w1b = w1_ref[j].astype(jnp.bfloat16)
        gate = lax.dot_general(
            xb, w1b[:I, :], dimension_numbers=(((1,), (1,)), ((), ())),
            preferred_element_type=jnp.float32)
        up = lax.dot_general(
            xb, w1b[I:, :], dimension_numbers=(((1,), (1,)), ((), ())),
            preferred_element_type=jnp.float32)
        h = (gate * jax.nn.sigmoid(gate)) * up
        hb = h.astype(jnp.bfloat16)
        w2b = w2_ref[j].astype(jnp.bfloat16)
        y = lax.dot_general(
            hb, w2b, dimension_numbers=(((1,), (1,)), ((), ())),
            preferred_element_type=jnp.float32)

        ee = e * EPB + j
        sel = jnp.where(ix_ref[...] == ee, sc_ref[...], 0.0)
        selp = jnp.concatenate(
            [sel, jnp.zeros((8 - K, T), jnp.float32)], axis=0)
        wcol = jnp.sum(jnp.transpose(selp), axis=1, keepdims=True)
        contrib = contrib + y * wcol

    @pl.when(e == 0)
    def _():
        out_ref[...] = contrib

    @pl.when(e > 0)
    def _():
        out_ref[...] += contrib


def _moe(x, w1, w2, scores_t, idx_t):
    return pl.pallas_call(
        _moe_body,
        grid=(E // EPB,),
        in_specs=[
            pl.BlockSpec((T, H), lambda e: (0, 0)),
            pl.BlockSpec((EPB, 2 * I, H), lambda e: (e, 0, 0)),
            pl.BlockSpec((EPB, H, I), lambda e: (e, 0, 0)),
            pl.BlockSpec((K, T), lambda e: (0, 0)),
            pl.BlockSpec((K, T), lambda e: (0, 0)),
        ],
        out_specs=pl.BlockSpec((T, H), lambda e: (0, 0)),
        out_shape=jax.ShapeDtypeStruct((T, H), jnp.float32),
        scratch_shapes=[pltpu.VMEM((T, H), jnp.bfloat16)],
    )(x, w1, w2, scores_t, idx_t)


def kernel(hidden_states, router_weight, w1, w2):
    orig_shape = hidden_states.shape
    x = hidden_states.reshape(-1, orig_shape[-1])
    logits_t = _router_logits(x, router_weight)
    scores_t, idx_t = _sc_routing(logits_t)
    out = _moe(x, w1, w2, scores_t, idx_t)
    return out.reshape(orig_shape)
